# baseline (speedup 1.0000x reference)
import functools
import numpy as np
import jax
import jax.numpy as jnp
from jax import lax
from jax.experimental import pallas as pl
from jax.experimental.pallas import tpu as pltpu
from jax.experimental.pallas import tpu_sc as plsc

_N, _C, _H, _W = 4, 96, 64, 64
_CP = 128
_HW = _H * _W
_RP = 1024
_NC, _NS, _L = 2, 16, 16
_RPT = _RP // (_NC * _NS)
_KC = _C // _L
_NEG = np.float32(-1e30)


def _sc_body(inp_hbm, b_hbm, x1_hbm, y1_hbm, x2_hbm, y2_hbm,
             out_hbm, bv, x1v, y1v, x2v, y2v,
             idx0, idx1, patch, sbuf, wbuf, outbuf, sem):
    wid = lax.axis_index("s") * _NC + lax.axis_index("c")
    base = wid * _RPT

    pltpu.sync_copy(b_hbm.at[pl.ds(base, _RPT)], bv)
    pltpu.sync_copy(x1_hbm.at[pl.ds(base, _RPT)], x1v)
    pltpu.sync_copy(y1_hbm.at[pl.ds(base, _RPT)], y1v)
    pltpu.sync_copy(x2_hbm.at[pl.ds(base, _RPT)], x2v)
    pltpu.sync_copy(y2_hbm.at[pl.ds(base, _RPT)], y2v)

    lane = lax.broadcasted_iota(jnp.int32, (_L,), 0)
    c96 = jnp.full((_L,), _C, jnp.int32)

    def roi_body(r, _):
        rv = jnp.full((_L,), r, jnp.int32)
        b_b = plsc.load_gather(bv, [rv])
        x1_b = plsc.load_gather(x1v, [rv])
        y1_b = plsc.load_gather(y1v, [rv])
        w_b = plsc.load_gather(x2v, [rv]) - x1_b
        h_b = plsc.load_gather(y2v, [rv]) - y1_b
        basev = (b_b * _H + y1_b) * _W + x1_b

        for j in range(16):
            idx_j = jnp.clip(basev + j * _W + lane, 0, _N * _HW - 1)
            if j < 8:
                idx0[pl.ds(j * _L, _L)] = idx_j
            else:
                idx1[pl.ds((j - 8) * _L, _L)] = idx_j
        cp0 = pltpu.async_copy(inp_hbm.at[idx0], patch.at[pl.ds(0, 128)], sem)
        cp1 = pltpu.async_copy(inp_hbm.at[idx1], patch.at[pl.ds(128, 128)], sem)
        cp0.wait()
        cp1.wait()

        lmask = lane < w_b
        mvec = jnp.full((_L,), _NEG)
        for j in range(16):
            s_j = plsc.load_gather(patch, [lane + j * _L, c96])
            sm_j = jnp.where(lmask & (h_b > j), s_j, _NEG)
            mvec = jnp.maximum(mvec, sm_j)
            sbuf[pl.ds(j * _L, _L)] = sm_j
        m = jnp.max(mvec)
        mb = jnp.full((_L,), m)
        dvec = jnp.zeros((_L,), jnp.float32)
        for j in range(16):
            e_j = jnp.exp(sbuf[pl.ds(j * _L, _L)] - mb)
            wbuf[pl.ds(j * _L, _L)] = e_j
            dvec = dvec + e_j
        denom = jnp.sum(dvec)
        db = jnp.full((_L,), denom)
        vvec = (w_b > 0) & (h_b > 0) & (db > 0.0)
        invb = jnp.where(vvec, 1.0 / jnp.where(vvec, db, 1.0),
                         jnp.float32(0.0))

        def jbody(j, accs):
            accs = list(accs)
            for l in range(16):
                p = j * 16 + l
                wb = plsc.load_gather(wbuf, [jnp.full((_L,), p, jnp.int32)])
                for k in range(_KC):
                    accs[k] = accs[k] + wb * patch[p, pl.ds(k * _L, _L)]
            return tuple(accs)

        accs = lax.fori_loop(
            0, 16, jbody,
            tuple(jnp.zeros((_L,), jnp.float32) for _ in range(_KC)))
        for k in range(_KC):
            outbuf[r, pl.ds(k * _L, _L)] = accs[k] * invb
        return ()

    lax.fori_loop(0, _RPT, roi_body, ())
    pltpu.sync_copy(outbuf, out_hbm.at[pl.ds(base, _RPT)])


@jax.jit
def _sc_call(inp_rows, b, x1, y1, x2, y2):
    mesh = plsc.VectorSubcoreMesh(
        core_axis_name="c", subcore_axis_name="s",
        num_cores=_NC, num_subcores=_NS)
    f = functools.partial(
        pl.kernel, mesh=mesh,
        compiler_params=pltpu.CompilerParams(needs_layout_passes=False),
        out_type=jax.ShapeDtypeStruct((_RP, _C), jnp.float32),
        scratch_types=[
            pltpu.VMEM((_RPT,), jnp.int32),
            pltpu.VMEM((_RPT,), jnp.int32),
            pltpu.VMEM((_RPT,), jnp.int32),
            pltpu.VMEM((_RPT,), jnp.int32),
            pltpu.VMEM((_RPT,), jnp.int32),
            pltpu.VMEM((128,), jnp.int32),
            pltpu.VMEM((128,), jnp.int32),
            pltpu.VMEM((256, _CP), jnp.float32),
            pltpu.VMEM((256,), jnp.float32),
            pltpu.VMEM((256,), jnp.float32),
            pltpu.VMEM((_RPT, _C), jnp.float32),
            pltpu.SemaphoreType.DMA,
        ])(_sc_body)
    return f(inp_rows, b, x1, y1, x2, y2)


def kernel(input, rois, score_map):
    N, C, H, W = input.shape
    R = rois.shape[0]

    inp_t = jnp.transpose(input, (0, 2, 3, 1)).reshape(N * H * W, C)
    score_col = score_map.reshape(N * H * W, 1)
    pad = jnp.zeros((N * H * W, _CP - C - 1), jnp.float32)
    inp_rows = jnp.concatenate([inp_t, score_col, pad], axis=1)

    ri = jnp.round(rois).astype(jnp.int32)
    ri = jnp.zeros((_RP, 5), jnp.int32).at[:R].set(ri)
    b, x1, y1, x2, y2 = [ri[:, i] for i in range(5)]

    out = _sc_call(inp_rows, b, x1, y1, x2, y2)
    return out[:R].reshape(R, C, 1, 1)


# speedup vs baseline: 1.0616x; 1.0616x over previous
import functools
import numpy as np
import jax
import jax.numpy as jnp
from jax import lax
from jax.experimental import pallas as pl
from jax.experimental.pallas import tpu as pltpu
from jax.experimental.pallas import tpu_sc as plsc

_N, _C, _H, _W = 4, 96, 64, 64
_CP = 128
_HW = _H * _W
_RP = 1024
_NC, _NS, _L = 2, 16, 16
_RPT = _RP // (_NC * _NS)
_KC = _C // _L
_NEG = np.float32(-1e30)


def _sc_body(inp_hbm, b_hbm, x1_hbm, y1_hbm, x2_hbm, y2_hbm,
             out_hbm, bv, x1v, y1v, x2v, y2v,
             idx0a, idx1a, idx0b, idx1b, patcha, patchb,
             sbuf, wbuf, outbuf, sema, semb):
    wid = lax.axis_index("s") * _NC + lax.axis_index("c")
    base = wid * _RPT

    pltpu.sync_copy(b_hbm.at[pl.ds(base, _RPT)], bv)
    pltpu.sync_copy(x1_hbm.at[pl.ds(base, _RPT)], x1v)
    pltpu.sync_copy(y1_hbm.at[pl.ds(base, _RPT)], y1v)
    pltpu.sync_copy(x2_hbm.at[pl.ds(base, _RPT)], x2v)
    pltpu.sync_copy(y2_hbm.at[pl.ds(base, _RPT)], y2v)

    lane = lax.broadcasted_iota(jnp.int32, (_L,), 0)
    c96 = jnp.full((_L,), _C, jnp.int32)

    def issue(r, idx0, idx1, patch, sem):
        rv = jnp.full((_L,), r, jnp.int32)
        b_b = plsc.load_gather(bv, [rv])
        x1_b = plsc.load_gather(x1v, [rv])
        y1_b = plsc.load_gather(y1v, [rv])
        basev = (b_b * _H + y1_b) * _W + x1_b
        for j in range(16):
            idx_j = jnp.clip(basev + j * _W + lane, 0, _N * _HW - 1)
            if j < 8:
                idx0[pl.ds(j * _L, _L)] = idx_j
            else:
                idx1[pl.ds((j - 8) * _L, _L)] = idx_j
        cp0 = pltpu.async_copy(inp_hbm.at[idx0], patch.at[pl.ds(0, 128)], sem)
        cp1 = pltpu.async_copy(inp_hbm.at[idx1], patch.at[pl.ds(128, 128)], sem)
        return cp0, cp1

    def wait(idx0, idx1, patch, sem):
        pltpu.make_async_copy(
            inp_hbm.at[idx0], patch.at[pl.ds(0, 128)], sem).wait()
        pltpu.make_async_copy(
            inp_hbm.at[idx1], patch.at[pl.ds(128, 128)], sem).wait()

    def compute(r, patch):
        rv = jnp.full((_L,), r, jnp.int32)
        x1_b = plsc.load_gather(x1v, [rv])
        y1_b = plsc.load_gather(y1v, [rv])
        w_b = plsc.load_gather(x2v, [rv]) - x1_b
        h_b = plsc.load_gather(y2v, [rv]) - y1_b

        lmask = lane < w_b
        mvec = jnp.full((_L,), _NEG)
        for j in range(16):
            s_j = plsc.load_gather(patch, [lane + j * _L, c96])
            sm_j = jnp.where(lmask & (h_b > j), s_j, _NEG)
            mvec = jnp.maximum(mvec, sm_j)
            sbuf[pl.ds(j * _L, _L)] = sm_j
        mb = jnp.full((_L,), jnp.max(mvec))

        dvec = jnp.zeros((_L,), jnp.float32)
        for j in range(16):
            e_j = jnp.exp(sbuf[pl.ds(j * _L, _L)] - mb)
            wbuf[pl.ds(j * _L, _L)] = e_j
            dvec = dvec + e_j

        def jbody(j, accs):
            accs = list(accs)
            for l in range(16):
                p = j * 16 + l
                wb = plsc.load_gather(wbuf, [jnp.full((_L,), p, jnp.int32)])
                for k in range(_KC):
                    accs[k] = accs[k] + wb * patch[p, pl.ds(k * _L, _L)]
            return tuple(accs)

        accs = lax.fori_loop(
            0, 16, jbody,
            tuple(jnp.zeros((_L,), jnp.float32) for _ in range(_KC)))

        db = jnp.full((_L,), jnp.sum(dvec))
        vvec = (w_b > 0) & (h_b > 0) & (db > 0.0)
        invb = jnp.where(vvec, 1.0 / jnp.where(vvec, db, 1.0),
                         jnp.float32(0.0))
        for k in range(_KC):
            outbuf[r, pl.ds(k * _L, _L)] = accs[k] * invb

    def pair_body(i, _):
        r0 = 2 * i
        cpa0, cpa1 = issue(r0, idx0a, idx1a, patcha, sema)
        cpb0, cpb1 = issue(r0 + 1, idx0b, idx1b, patchb, semb)
        cpa0.wait()
        cpa1.wait()
        compute(r0, patcha)
        cpb0.wait()
        cpb1.wait()
        compute(r0 + 1, patchb)
        return ()

    lax.fori_loop(0, _RPT // 2, pair_body, ())

    pltpu.sync_copy(outbuf, out_hbm.at[pl.ds(base, _RPT)])


@jax.jit
def _sc_call(inp_rows, b, x1, y1, x2, y2):
    mesh = plsc.VectorSubcoreMesh(
        core_axis_name="c", subcore_axis_name="s",
        num_cores=_NC, num_subcores=_NS)
    f = functools.partial(
        pl.kernel, mesh=mesh,
        compiler_params=pltpu.CompilerParams(needs_layout_passes=False),
        out_type=jax.ShapeDtypeStruct((_RP, _C), jnp.float32),
        scratch_types=[
            pltpu.VMEM((_RPT,), jnp.int32),
            pltpu.VMEM((_RPT,), jnp.int32),
            pltpu.VMEM((_RPT,), jnp.int32),
            pltpu.VMEM((_RPT,), jnp.int32),
            pltpu.VMEM((_RPT,), jnp.int32),
            pltpu.VMEM((128,), jnp.int32),
            pltpu.VMEM((128,), jnp.int32),
            pltpu.VMEM((128,), jnp.int32),
            pltpu.VMEM((128,), jnp.int32),
            pltpu.VMEM((256, _CP), jnp.float32),
            pltpu.VMEM((256, _CP), jnp.float32),
            pltpu.VMEM((256,), jnp.float32),
            pltpu.VMEM((256,), jnp.float32),
            pltpu.VMEM((_RPT, _C), jnp.float32),
            pltpu.SemaphoreType.DMA,
            pltpu.SemaphoreType.DMA,
        ])(_sc_body)
    return f(inp_rows, b, x1, y1, x2, y2)


def kernel(input, rois, score_map):
    N, C, H, W = input.shape
    R = rois.shape[0]

    inp_t = jnp.transpose(input, (0, 2, 3, 1)).reshape(N * H * W, C)
    score_col = score_map.reshape(N * H * W, 1)
    pad = jnp.zeros((N * H * W, _CP - C - 1), jnp.float32)
    inp_rows = jnp.concatenate([inp_t, score_col, pad], axis=1)

    ri = jnp.round(rois).astype(jnp.int32)
    ri = jnp.zeros((_RP, 5), jnp.int32).at[:R].set(ri)
    b, x1, y1, x2, y2 = [ri[:, i] for i in range(5)]

    out = _sc_call(inp_rows, b, x1, y1, x2, y2)
    return out[:R].reshape(R, C, 1, 1)
